# Initial kernel scaffold; baseline (speedup 1.0000x reference)
import jax
import jax.numpy as jnp
from jax.experimental import pallas as pl


def kernel(x, mesh_edge_attr, world_edge_attr, params, mesh_edge_index, world_edge_index):
    raise NotImplementedError("write your pallas kernel here")


# speedup vs baseline: 1.2552x; 1.2552x over previous
import functools

import jax
import jax.numpy as jnp
from jax import lax
from jax.experimental import pallas as pl
from jax.experimental.pallas import tpu as pltpu
from jax.experimental.pallas import tpu_sc as plsc

H = 128
N = 19674
NH = 9856
NPAD = 2 * NH
NHT = 9984
ZSPAN = NHT // 16
WSPAN = NH // 16
EM = 118044
EW = 19674
EMP = 118784
EWP = 20480
NC, NS = 2, 16
NW = NC * NS
OUT_ROWS = 4981
C = 128

_f32 = jnp.float32


@functools.cache
def _sc_mesh():
    return plsc.VectorSubcoreMesh(
        core_axis_name="c", subcore_axis_name="s",
        num_cores=NC, num_subcores=NS)


def _dot(a, b):
    return jnp.dot(a, b, preferred_element_type=_f32,
                   precision=lax.Precision.HIGHEST)


def _leaky(x):
    return jnp.where(x > 0, x, 0.2 * x)


def _ln(x, g, b):
    mu = jnp.mean(x, axis=-1, keepdims=True)
    var = jnp.mean((x - mu) ** 2, axis=-1, keepdims=True)
    return (x - mu) * lax.rsqrt(var + 1e-5) * g + b


def _sc_gather(tam, tbm, taw, tbw, sm, rm, sw, rw):
    @functools.partial(
        pl.kernel,
        out_type=(jax.ShapeDtypeStruct((EMP, H), _f32),
                  jax.ShapeDtypeStruct((EMP, H), _f32),
                  jax.ShapeDtypeStruct((EWP, H), _f32),
                  jax.ShapeDtypeStruct((EWP, H), _f32)),
        mesh=_sc_mesh(),
        scratch_types=[pltpu.VMEM((C,), jnp.int32),
                       pltpu.VMEM((C, H), _f32),
                       pltpu.SemaphoreType.DMA],
    )
    def k(tam_h, tbm_h, taw_h, tbw_h, sm_h, rm_h, sw_h, rw_h,
          gam_h, gbm_h, gaw_h, gbw_h, idx_v, rows_v, sem):
        wid = lax.axis_index("s") * NC + lax.axis_index("c")

        def one(tab_h, idx_h, out_h, epad):
            per_tile = epad // NW
            base = wid * per_tile

            def body(j, carry):
                off = base + j * C
                pltpu.sync_copy(idx_h.at[pl.ds(off, C)], idx_v)
                pltpu.async_copy(tab_h.at[idx_v], rows_v, sem).wait()
                pltpu.sync_copy(rows_v, out_h.at[pl.ds(off, C)])
                return carry

            lax.fori_loop(0, per_tile // C, body, 0)

        one(tam_h, sm_h, gam_h, EMP)
        one(tbm_h, rm_h, gbm_h, EMP)
        one(taw_h, sw_h, gaw_h, EWP)
        one(tbw_h, rw_h, gbw_h, EWP)

    return k(tam, tbm, taw, tbw, sm, rm, sw, rw)


def _sc_scatter(vals_m, vals_w, lidx_m, lidx_w, zrows):
    @functools.partial(
        pl.kernel,
        out_type=(jax.ShapeDtypeStruct((NPAD, H), _f32),
                  jax.ShapeDtypeStruct((NPAD, H), _f32)),
        mesh=_sc_mesh(),
        scratch_types=[pltpu.VMEM_SHARED((NHT, H), _f32),
                       pltpu.VMEM((C, H), _f32),
                       pltpu.VMEM((C,), jnp.int32),
                       pltpu.SemaphoreType.DMA],
    )
    def k(vm_h, vw_h, lim_h, liw_h, zr_h, aggm_h, aggw_h, table, rows_v,
          idx_v, sem):
        ci = lax.axis_index("c")
        t = lax.axis_index("s")

        def phase(val_h, li_h, epad, out_h):
            pltpu.sync_copy(zr_h, table.at[pl.ds(t * ZSPAN, ZSPAN)])
            plsc.subcore_barrier()
            per_tile = epad // NS

            def body(j, carry):
                off = t * per_tile + j * C
                pltpu.sync_copy(val_h.at[pl.ds(off, C)], rows_v)
                pltpu.sync_copy(li_h.at[pl.ds(ci * epad + off, C)], idx_v)
                pltpu.sync_copy(rows_v, table.at[idx_v], add=True)
                return carry

            lax.fori_loop(0, per_tile // C, body, 0)
            plsc.subcore_barrier()
            pltpu.sync_copy(table.at[pl.ds(t * WSPAN, WSPAN)],
                            out_h.at[pl.ds(ci * NH + t * WSPAN, WSPAN)])
            plsc.subcore_barrier()

        phase(vm_h, lim_h, EMP, aggm_h)
        phase(vw_h, liw_h, EWP, aggw_h)

    return k(vals_m, vals_w, lidx_m, lidx_w, zrows)


def _sc_counts(lidx_m, lidx_w, zrows, orows):
    @functools.partial(
        pl.kernel,
        out_type=(jax.ShapeDtypeStruct((NPAD, H), _f32),
                  jax.ShapeDtypeStruct((NPAD, H), _f32)),
        mesh=_sc_mesh(),
        scratch_types=[pltpu.VMEM_SHARED((NHT, H), _f32),
                       pltpu.VMEM((C, H), _f32),
                       pltpu.VMEM((C,), jnp.int32),
                       pltpu.SemaphoreType.DMA],
    )
    def k(lim_h, liw_h, zr_h, or_h, cm_h, cw_h, table, ones_v, idx_v, sem):
        ci = lax.axis_index("c")
        t = lax.axis_index("s")
        pltpu.sync_copy(or_h, ones_v)

        def phase(li_h, epad, out_h):
            pltpu.sync_copy(zr_h, table.at[pl.ds(t * ZSPAN, ZSPAN)])
            plsc.subcore_barrier()
            per_tile = epad // NS

            def body(j, carry):
                off = t * per_tile + j * C
                pltpu.sync_copy(li_h.at[pl.ds(ci * epad + off, C)], idx_v)
                pltpu.sync_copy(ones_v, table.at[idx_v], add=True)
                return carry

            lax.fori_loop(0, per_tile // C, body, 0)
            plsc.subcore_barrier()
            pltpu.sync_copy(table.at[pl.ds(t * WSPAN, WSPAN)],
                            out_h.at[pl.ds(ci * NH + t * WSPAN, WSPAN)])
            plsc.subcore_barrier()

        phase(lim_h, EMP, cm_h)
        phase(liw_h, EWP, cw_h)

    return k(lidx_m, lidx_w, zrows, orows)


def _row_spec(r):
    return pl.BlockSpec((r, H), lambda i: (i, 0))


def _full_spec(shape):
    nd = len(shape)
    return pl.BlockSpec(shape, lambda i: (0,) * nd)


def _mlp4(x, w1, b1, w2, b2, w3, b3, w4):
    x = _leaky(_dot(x, w1) + b1)
    x = _leaky(_dot(x, w2) + b2)
    x = _leaky(_dot(x, w3) + b3)
    return _dot(x, w4)


def _enc_body(x, w1, b1, w2, b2, w3, b3, w4, g, bb, o):
    y = _mlp4(x[...], w1[...], b1[...], w2[...], b2[...], w3[...], b3[...],
              w4[...])
    o[...] = _ln(y, g[...], bb[...])


def _edge_body(ga, gb, e, w1c, b1, w2, b2, w3, b3, w4, g, bb, o):
    ev = e[...]
    x = ga[...] + gb[...] + _dot(ev, w1c[...]) + b1[...]
    x = _leaky(x)
    x = _leaky(_dot(x, w2[...]) + b2[...])
    x = _leaky(_dot(x, w3[...]) + b3[...])
    x = _dot(x, w4[...])
    o[...] = ev + _ln(x, g[...], bb[...])


def _node_body(h, sm, sw, cm, cw, v1a, v1b, v1c, b1, w2, b2, w3, b3, w4, g,
               bb, wam, wbm, waw, wbw, ho, tam, tbm, taw, tbw):
    hv = h[...]
    am = sm[...] / jnp.maximum(cm[...], 1.0)
    aw = sw[...] / jnp.maximum(cw[...], 1.0)
    x = _dot(hv, v1a[...]) + _dot(am, v1b[...]) + _dot(aw, v1c[...]) + b1[...]
    x = _leaky(x)
    x = _leaky(_dot(x, w2[...]) + b2[...])
    x = _leaky(_dot(x, w3[...]) + b3[...])
    x = _dot(x, w4[...])
    hn = hv + _ln(x, g[...], bb[...])
    ho[...] = hn
    tam[...] = _dot(hn, wam[...])
    tbm[...] = _dot(hn, wbm[...])
    taw[...] = _dot(hn, waw[...])
    tbw[...] = _dot(hn, wbw[...])


def _proj_body(h, wam, wbm, waw, wbw, tam, tbm, taw, tbw):
    hv = h[...]
    tam[...] = _dot(hv, wam[...])
    tbm[...] = _dot(hv, wbm[...])
    taw[...] = _dot(hv, waw[...])
    tbw[...] = _dot(hv, wbw[...])


def _dec_body(h, w1, b1, w2, b2, w3, b3, w4, o):
    o[...] = _mlp4(h[...], w1[...], b1[...], w2[...], b2[...], w3[...],
                   b3[...], w4[...])


def _mlpln_weights(p):
    mlp = p["mlp"]
    return (mlp[0]["W"], mlp[0]["b"].reshape(1, H),
            mlp[1]["W"], mlp[1]["b"].reshape(1, H),
            mlp[2]["W"], mlp[2]["b"].reshape(1, H),
            mlp[3]["W"],
            p["g"].reshape(1, H), p["b"].reshape(1, H))


def _enc_call(x, p, r=1232):
    w = _mlpln_weights(p)
    rows = x.shape[0]
    return pl.pallas_call(
        _enc_body,
        grid=(pl.cdiv(rows, r),),
        in_specs=[pl.BlockSpec((r, x.shape[1]), lambda i: (i, 0))]
        + [_full_spec(a.shape) for a in w],
        out_specs=_row_spec(r),
        out_shape=jax.ShapeDtypeStruct((rows, H), _f32),
    )(x, *w)


def _edge_call(ga, gb, e, p, r=1536):
    w1, b1, w2, b2, w3, b3, w4, g, bb = _mlpln_weights(p)
    w = (w1[2 * H:], b1, w2, b2, w3, b3, w4, g, bb)
    rows = e.shape[0]
    return pl.pallas_call(
        _edge_body,
        grid=(pl.cdiv(rows, r),),
        in_specs=[_row_spec(r)] * 3 + [_full_spec(a.shape) for a in w],
        out_specs=_row_spec(r),
        out_shape=jax.ShapeDtypeStruct((rows, H), _f32),
    )(ga, gb, e, *w)


def _proj_weights(blk):
    wm = blk["edge_mesh"]["mlp"][0]["W"]
    ww = blk["edge_world"]["mlp"][0]["W"]
    return wm[:H], wm[H:2 * H], ww[:H], ww[H:2 * H]


def _node_call(h, sm, sw, cm, cw, p, nxt, r=1232):
    w1, b1, w2, b2, w3, b3, w4, g, bb = _mlpln_weights(p)
    w = (w1[:H], w1[H:2 * H], w1[2 * H:], b1, w2, b2, w3, b3, w4, g, bb)
    w = w + _proj_weights(nxt)
    out = jax.ShapeDtypeStruct((NPAD, H), _f32)
    return pl.pallas_call(
        _node_body,
        grid=(pl.cdiv(NPAD, r),),
        in_specs=[_row_spec(r)] * 5 + [_full_spec(a.shape) for a in w],
        out_specs=[_row_spec(r)] * 5,
        out_shape=[out] * 5,
    )(h, sm, sw, cm, cw, *w)


def _proj_call(h, blk, r=1232):
    w = _proj_weights(blk)
    out = jax.ShapeDtypeStruct((NPAD, H), _f32)
    return pl.pallas_call(
        _proj_body,
        grid=(pl.cdiv(NPAD, r),),
        in_specs=[_row_spec(r)] + [_full_spec(a.shape) for a in w],
        out_specs=[_row_spec(r)] * 4,
        out_shape=[out] * 4,
    )(h, *w)


def _dec_call(h, dec, r=1232):
    w = (dec[0]["W"], dec[0]["b"].reshape(1, H),
         dec[1]["W"], dec[1]["b"].reshape(1, H),
         dec[2]["W"], dec[2]["b"].reshape(1, H),
         dec[3]["W"])
    return pl.pallas_call(
        _dec_body,
        grid=(pl.cdiv(NPAD, r),),
        in_specs=[_row_spec(r)] + [_full_spec(a.shape) for a in w],
        out_specs=pl.BlockSpec((r, 4), lambda i: (i, 0)),
        out_shape=jax.ShapeDtypeStruct((NPAD, 4), _f32),
    )(h, *w)


def _pad_rows(a, n):
    return jnp.pad(a, ((0, n - a.shape[0]), (0, 0)))


def _pad_idx(v, n):
    return jnp.pad(v.astype(jnp.int32), (0, n - v.shape[0]))


def _local_idx(dst, epad):
    dst = dst.astype(jnp.int32)
    parts = []
    for c in range(NC):
        li = dst - c * NH
        li = jnp.where((li >= 0) & (li < NH), li, NH)
        parts.append(jnp.pad(li, (0, epad - dst.shape[0]), constant_values=NH))
    return jnp.concatenate(parts)


def kernel(x, mesh_edge_attr, world_edge_attr, params, mesh_edge_index,
           world_edge_index):
    xp = _pad_rows(x, NPAD)
    mep = _pad_rows(mesh_edge_attr, EMP)
    wep = _pad_rows(world_edge_attr, EWP)
    sm = _pad_idx(mesh_edge_index[0], EMP)
    rm = _pad_idx(mesh_edge_index[1], EMP)
    sw = _pad_idx(world_edge_index[0], EWP)
    rw = _pad_idx(world_edge_index[1], EWP)
    lidx_m = _local_idx(mesh_edge_index[1], EMP)
    lidx_w = _local_idx(world_edge_index[1], EWP)
    zrows = jnp.zeros((ZSPAN, H), _f32)
    orows = jnp.ones((C, H), _f32)

    h = _enc_call(xp, params["enc_node"])
    me = _enc_call(mep, params["enc_mesh"])
    we = _enc_call(wep, params["enc_world"])
    cm, cw = _sc_counts(lidx_m, lidx_w, zrows, orows)

    blocks = params["blocks"]
    tabs = _proj_call(h, blocks[0])
    for i in range(len(blocks)):
        blk = blocks[i]
        gam, gbm, gaw, gbw = _sc_gather(*tabs, sm, rm, sw, rw)
        me = _edge_call(gam, gbm, me, blk["edge_mesh"])
        we = _edge_call(gaw, gbw, we, blk["edge_world"])
        summ, sumw = _sc_scatter(me, we, lidx_m, lidx_w, zrows)
        nxt = blocks[i + 1] if i + 1 < len(blocks) else blocks[0]
        h, *tabs = _node_call(h, summ, sumw, cm, cw, blk["node"], nxt)

    out = _dec_call(h, params["dec"])
    return out[:OUT_ROWS]
